# Initial kernel scaffold; baseline (speedup 1.0000x reference)
import jax
import jax.numpy as jnp
from jax.experimental import pallas as pl


def kernel(x, token_table, pos_table):
    raise NotImplementedError("write your pallas kernel here")


# speedup vs baseline: 3.7179x; 3.7179x over previous
import functools

import jax
import jax.numpy as jnp
from jax import lax
from jax.experimental import pallas as pl
from jax.experimental.pallas import tpu as pltpu
from jax.experimental.pallas import tpu_sc as plsc

NC = 2
NS = 16
NW = NC * NS
LANES = 16

VOCAB = 100000
MAXLEN = 200
EMBED = 64
BATCH = 4096

TOK = BATCH * MAXLEN
TPW = TOK // NW
CHUNK = 800
NCHUNK = TPW // CHUNK

assert TOK % NW == 0 and TPW % CHUNK == 0 and CHUNK % MAXLEN == 0
SEGS = CHUNK // MAXLEN


def _emb_body(x_hbm, tok_hbm, pos_hbm, out_hbm, idx_v, rows_v, pos_v, sem):
    wid = lax.axis_index("s") * NC + lax.axis_index("c")
    pltpu.sync_copy(pos_hbm, pos_v)

    @pl.loop(0, NCHUNK)
    def _chunk(g):
        base = wid * TPW + g * CHUNK
        pltpu.sync_copy(x_hbm.at[pl.ds(base, CHUNK)], idx_v)
        pltpu.async_copy(tok_hbm.at[idx_v], rows_v, sem).wait()

        def _add(j, carry):
            for k in range(EMBED // LANES):
                p = pos_v[j, pl.ds(k * LANES, LANES)]
                for s in range(SEGS):
                    plsc.addupdate(rows_v.at[s * MAXLEN + j, pl.ds(k * LANES, LANES)], p)
            return carry

        lax.fori_loop(0, MAXLEN, _add, None)
        pltpu.sync_copy(rows_v, out_hbm.at[pl.ds(base, CHUNK)])


_emb = functools.partial(
    pl.kernel,
    out_type=jax.ShapeDtypeStruct((TOK, EMBED), jnp.float32),
    mesh=plsc.VectorSubcoreMesh(core_axis_name="c", subcore_axis_name="s"),
    scratch_types=[
        pltpu.VMEM((CHUNK,), jnp.int32),
        pltpu.VMEM((CHUNK, EMBED), jnp.float32),
        pltpu.VMEM((MAXLEN, EMBED), jnp.float32),
        pltpu.SemaphoreType.DMA,
    ],
    compiler_params=pltpu.CompilerParams(use_tc_tiling_on_sc=False),
)(_emb_body)


def kernel(x, token_table, pos_table):
    b, l = x.shape
    xf = x.reshape(-1).astype(jnp.int32)
    out = _emb(xf, token_table, pos_table)
    return out.reshape(b, l, EMBED)


# speedup vs baseline: 4.1294x; 1.1107x over previous
import functools

import jax
import jax.numpy as jnp
from jax import lax
from jax.experimental import pallas as pl
from jax.experimental.pallas import tpu as pltpu
from jax.experimental.pallas import tpu_sc as plsc

NC = 2
NS = 16
NW = NC * NS
LANES = 16

VOCAB = 100000
MAXLEN = 200
EMBED = 64
BATCH = 4096

TOK = BATCH * MAXLEN
TPW = TOK // NW
CHUNK = 800
NCHUNK = TPW // CHUNK

assert TOK % NW == 0 and TPW % CHUNK == 0 and CHUNK % MAXLEN == 0
assert NCHUNK % 2 == 0 and NCHUNK >= 4
SEGS = CHUNK // MAXLEN


def _emb_body(x_hbm, tok_hbm, pos_hbm, out_hbm,
              idx0, idx1, rows0, rows1, pos_v,
              gsem0, gsem1, osem0, osem1):
    idx = (idx0, idx1)
    rows = (rows0, rows1)
    gsem = (gsem0, gsem1)
    osem = (osem0, osem1)
    wid = lax.axis_index("s") * NC + lax.axis_index("c")
    base0 = wid * TPW
    pltpu.sync_copy(pos_hbm, pos_v)

    def start_gather(c, b):
        pltpu.sync_copy(x_hbm.at[pl.ds(base0 + c * CHUNK, CHUNK)], idx[b])
        pltpu.async_copy(tok_hbm.at[idx[b]], rows[b], gsem[b])

    def wait_gather(b):
        pltpu.make_async_copy(tok_hbm.at[idx[b]], rows[b], gsem[b]).wait()

    def add_pos(b):
        rv = rows[b]

        def _add(j, carry):
            for k in range(EMBED // LANES):
                p = pos_v[j, pl.ds(k * LANES, LANES)]
                for s in range(SEGS):
                    plsc.addupdate(rv.at[s * MAXLEN + j, pl.ds(k * LANES, LANES)], p)
            return carry

        lax.fori_loop(0, MAXLEN, _add, None)

    def start_scatter(c, b):
        pltpu.async_copy(rows[b], out_hbm.at[pl.ds(base0 + c * CHUNK, CHUNK)], osem[b])

    def wait_scatter(b):
        pltpu.make_async_copy(rows[b], out_hbm.at[pl.ds(0, CHUNK)], osem[b]).wait()

    start_gather(0, 0)
    wait_gather(0)
    start_gather(1, 1)
    add_pos(0)
    start_scatter(0, 0)

    @pl.loop(0, (NCHUNK - 2) // 2)
    def _steady(t):
        for b in (1, 0):
            c = 1 + 2 * t + (0 if b == 1 else 1)
            wait_gather(b)
            ob = 1 - b
            wait_scatter(ob)
            start_gather(c + 1, ob)
            add_pos(b)
            start_scatter(c, b)

    wait_gather(1)
    add_pos(1)
    start_scatter(NCHUNK - 1, 1)
    wait_scatter(0)
    wait_scatter(1)


_emb = functools.partial(
    pl.kernel,
    out_type=jax.ShapeDtypeStruct((TOK, EMBED), jnp.float32),
    mesh=plsc.VectorSubcoreMesh(core_axis_name="c", subcore_axis_name="s"),
    scratch_types=[
        pltpu.VMEM((CHUNK,), jnp.int32),
        pltpu.VMEM((CHUNK,), jnp.int32),
        pltpu.VMEM((CHUNK, EMBED), jnp.float32),
        pltpu.VMEM((CHUNK, EMBED), jnp.float32),
        pltpu.VMEM((MAXLEN, EMBED), jnp.float32),
        pltpu.SemaphoreType.DMA,
        pltpu.SemaphoreType.DMA,
        pltpu.SemaphoreType.DMA,
        pltpu.SemaphoreType.DMA,
    ],
    compiler_params=pltpu.CompilerParams(use_tc_tiling_on_sc=False),
)(_emb_body)


def kernel(x, token_table, pos_table):
    b, l = x.shape
    xf = x.reshape(-1).astype(jnp.int32)
    out = _emb(xf, token_table, pos_table)
    return out.reshape(b, l, EMBED)


# speedup vs baseline: 4.1391x; 1.0023x over previous
import functools

import jax
import jax.numpy as jnp
from jax import lax
from jax.experimental import pallas as pl
from jax.experimental.pallas import tpu as pltpu
from jax.experimental.pallas import tpu_sc as plsc

NC = 2
NS = 16
NW = NC * NS
LANES = 16

VOCAB = 100000
MAXLEN = 200
EMBED = 64
BATCH = 4096

RPW = BATCH // NW
SEGS = 4
NCHUNK = RPW // SEGS

assert BATCH % NW == 0 and RPW % SEGS == 0
assert NCHUNK % 2 == 0 and NCHUNK >= 4


def _emb_body(x_hbm, tok_hbm, pos_hbm, out_hbm,
              idx0, idx1, rows0, rows1, pos_v,
              gsem0, gsem1, osem0, osem1):
    idx = (idx0, idx1)
    rows = (rows0, rows1)
    gsem = (gsem0, gsem1)
    osem = (osem0, osem1)
    wid = lax.axis_index("s") * NC + lax.axis_index("c")
    row0 = wid * RPW
    pltpu.sync_copy(pos_hbm, pos_v)

    def start_gather(c, b):
        pltpu.sync_copy(x_hbm.at[pl.ds(row0 + c * SEGS, SEGS)], idx[b])
        for s in range(SEGS):
            pltpu.async_copy(tok_hbm.at[idx[b].at[s]], rows[b].at[s], gsem[b])

    def wait_gather(b):
        for s in range(SEGS):
            pltpu.make_async_copy(tok_hbm.at[idx[b].at[s]], rows[b].at[s], gsem[b]).wait()

    def add_pos(b):
        rv = rows[b]

        def _add(j, carry):
            for k in range(EMBED // LANES):
                p = pos_v[j, pl.ds(k * LANES, LANES)]
                for s in range(SEGS):
                    plsc.addupdate(rv.at[s, j, pl.ds(k * LANES, LANES)], p)
            return carry

        lax.fori_loop(0, MAXLEN, _add, None)

    def start_scatter(c, b):
        pltpu.async_copy(rows[b], out_hbm.at[pl.ds(row0 + c * SEGS, SEGS)], osem[b])

    def wait_scatter(b):
        pltpu.make_async_copy(rows[b], out_hbm.at[pl.ds(row0, SEGS)], osem[b]).wait()

    start_gather(0, 0)
    wait_gather(0)
    start_gather(1, 1)
    add_pos(0)
    start_scatter(0, 0)

    @pl.loop(0, (NCHUNK - 2) // 2)
    def _steady(t):
        for b in (1, 0):
            c = 1 + 2 * t + (0 if b == 1 else 1)
            wait_gather(b)
            ob = 1 - b
            wait_scatter(ob)
            start_gather(c + 1, ob)
            add_pos(b)
            start_scatter(c, b)

    wait_gather(1)
    add_pos(1)
    start_scatter(NCHUNK - 1, 1)
    wait_scatter(0)
    wait_scatter(1)


_emb = functools.partial(
    pl.kernel,
    out_type=jax.ShapeDtypeStruct((BATCH, MAXLEN, EMBED), jnp.float32),
    mesh=plsc.VectorSubcoreMesh(core_axis_name="c", subcore_axis_name="s"),
    scratch_types=[
        pltpu.VMEM((SEGS, MAXLEN), jnp.int32),
        pltpu.VMEM((SEGS, MAXLEN), jnp.int32),
        pltpu.VMEM((SEGS, MAXLEN, EMBED), jnp.float32),
        pltpu.VMEM((SEGS, MAXLEN, EMBED), jnp.float32),
        pltpu.VMEM((MAXLEN, EMBED), jnp.float32),
        pltpu.SemaphoreType.DMA,
        pltpu.SemaphoreType.DMA,
        pltpu.SemaphoreType.DMA,
        pltpu.SemaphoreType.DMA,
    ],
    compiler_params=pltpu.CompilerParams(use_tc_tiling_on_sc=False),
)(_emb_body)


def kernel(x, token_table, pos_table):
    return _emb(x.astype(jnp.int32), token_table, pos_table)


# speedup vs baseline: 4.2721x; 1.0321x over previous
import functools

import jax
import jax.numpy as jnp
from jax import lax
from jax.experimental import pallas as pl
from jax.experimental.pallas import tpu as pltpu
from jax.experimental.pallas import tpu_sc as plsc

NC = 2
NS = 16
NW = NC * NS
LANES = 16

VOCAB = 100000
MAXLEN = 200
EMBED = 64
EPAD = 128
BATCH = 4096

RPW = BATCH // NW
NCHUNK = RPW
assert BATCH % NW == 0 and NCHUNK % 2 == 0


def _emb_body(x_hbm, tok_hbm, pos_hbm, out_hbm,
              idx0, idx1, rows0, rows1, cmp0, cmp1, pos_v,
              gsem0, gsem1, osem0, osem1):
    idx = (idx0, idx1)
    rows = (rows0, rows1)
    cmp_ = (cmp0, cmp1)
    gsem = (gsem0, gsem1)
    osem = (osem0, osem1)
    wid = lax.axis_index("s") * NC + lax.axis_index("c")
    row0 = wid * RPW
    pltpu.sync_copy(pos_hbm, pos_v)

    def start_gather(c, b):
        pltpu.sync_copy(x_hbm.at[pl.ds((row0 + c) * MAXLEN, MAXLEN)], idx[b])
        pltpu.async_copy(tok_hbm.at[idx[b]], rows[b], gsem[b])

    def wait_gather(b):
        pltpu.make_async_copy(tok_hbm.at[idx[b]], rows[b], gsem[b]).wait()

    def add_pos(b):
        rv, cv = rows[b], cmp_[b]

        def _add(j, carry):
            for k in range(EMBED // LANES):
                p = pos_v[j, pl.ds(k * LANES, LANES)]
                t = rv[j, pl.ds(k * LANES, LANES)]
                cv[j, pl.ds(k * LANES, LANES)] = p + t
            return carry

        lax.fori_loop(0, MAXLEN, _add, None)

    def start_scatter(c, b):
        pltpu.async_copy(cmp_[b], out_hbm.at[row0 + c], osem[b])

    def wait_scatter(b):
        pltpu.make_async_copy(cmp_[b], out_hbm.at[row0], osem[b]).wait()

    start_gather(0, 0)
    wait_gather(0)
    start_gather(1, 1)
    add_pos(0)
    start_scatter(0, 0)

    @pl.loop(0, (NCHUNK - 2) // 2)
    def _steady(t):
        for b in (1, 0):
            c = 1 + 2 * t + (0 if b == 1 else 1)
            wait_gather(b)
            ob = 1 - b
            wait_scatter(ob)
            start_gather(c + 1, ob)
            add_pos(b)
            start_scatter(c, b)

    wait_gather(1)
    add_pos(1)
    start_scatter(NCHUNK - 1, 1)
    wait_scatter(0)
    wait_scatter(1)


_emb = functools.partial(
    pl.kernel,
    out_type=jax.ShapeDtypeStruct((BATCH, MAXLEN, EMBED), jnp.float32),
    mesh=plsc.VectorSubcoreMesh(core_axis_name="c", subcore_axis_name="s"),
    scratch_types=[
        pltpu.VMEM((MAXLEN,), jnp.int32),
        pltpu.VMEM((MAXLEN,), jnp.int32),
        pltpu.VMEM((MAXLEN, EPAD), jnp.float32),
        pltpu.VMEM((MAXLEN, EPAD), jnp.float32),
        pltpu.VMEM((MAXLEN, EMBED), jnp.float32),
        pltpu.VMEM((MAXLEN, EMBED), jnp.float32),
        pltpu.VMEM((MAXLEN, EPAD), jnp.float32),
        pltpu.SemaphoreType.DMA,
        pltpu.SemaphoreType.DMA,
        pltpu.SemaphoreType.DMA,
        pltpu.SemaphoreType.DMA,
    ],
    compiler_params=pltpu.CompilerParams(use_tc_tiling_on_sc=True),
)(_emb_body)


def kernel(x, token_table, pos_table):
    xf = x.reshape(-1).astype(jnp.int32)
    tok_pad = jnp.pad(token_table, ((0, 0), (0, EPAD - EMBED)))
    pos_pad = jnp.pad(pos_table, ((0, 0), (0, EPAD - EMBED)))
    return _emb(xf, tok_pad, pos_pad)
